# baseline (speedup 1.0000x reference)
import functools

import jax
import jax.numpy as jnp
from jax import lax
from jax.experimental import pallas as pl
from jax.experimental.pallas import tpu as pltpu
from jax.experimental.pallas import tpu_sc as plsc

NUM_CONVS = 4
D = 128
N_NODES = 10000
N_EDGES = 320000
EH = N_EDGES // 2
NC, NS = 2, 16
NW = NC * NS

G_CH = 96
G_K = 4
S_CH = 128
ZR = 624
ZTAIL = N_NODES - NS * ZR

_mesh = plsc.VectorSubcoreMesh(
    core_axis_name="c", subcore_axis_name="s", num_cores=NC, num_subcores=NS)


GPW = 2 * EH // NW
G_GROUP = G_CH * G_K
G_NGRP = GPW // G_GROUP
G_TAIL = GPW - G_NGRP * G_GROUP

@functools.partial(
    pl.kernel,
    out_type=jax.ShapeDtypeStruct((2 * EH, D), jnp.float32),
    mesh=_mesh,
    scratch_types=[
        pltpu.VMEM((GPW,), jnp.int32),
        [pltpu.VMEM((G_CH, D), jnp.float32)] * G_K,
        [pltpu.VMEM((G_CH, D), jnp.float32)] * G_K,
        pltpu.SemaphoreType.DMA,
        pltpu.SemaphoreType.DMA,
        pltpu.SemaphoreType.DMA,
        pltpu.SemaphoreType.DMA,
    ],
)
def _gather(table, idx, out, idx_v, bufa, bufb, gsa, gsb, wsa, wsb):
    bufs = (bufa, bufb)
    gs = (gsa, gsb)
    ws = (wsa, wsb)
    wid = lax.axis_index("s") * NC + lax.axis_index("c")
    base = pl.multiple_of(wid * GPW, GPW)
    pltpu.sync_copy(idx.at[pl.ds(base, GPW)], idx_v)

    def fire_g(g, par):
        goff = pl.multiple_of(g * G_GROUP, G_GROUP)
        for k in range(G_K):
            pltpu.async_copy(
                table.at[idx_v.at[pl.ds(goff + k * G_CH, G_CH)]],
                bufs[par][k], gs[par])

    def drain_g(par):
        for k in range(G_K):
            pltpu.make_async_copy(
                table.at[idx_v.at[pl.ds(0, G_CH)]], bufs[par][k],
                gs[par]).wait()

    def fire_w(g, par):
        goff = pl.multiple_of(g * G_GROUP, G_GROUP)
        for k in range(G_K):
            pltpu.async_copy(
                bufs[par][k],
                out.at[pl.ds(base + goff + k * G_CH, G_CH)], ws[par])

    def drain_w(par):
        for k in range(G_K):
            pltpu.make_async_copy(
                bufs[par][k], out.at[pl.ds(base, G_CH)], ws[par]).wait()

    fire_g(0, 0)

    def pair(p, carry):
        g = 2 * p
        for b in range(2):
            fire_g(g + b + 1, 1 - b)
            drain_g(b)
            fire_w(g + b, b)
            drain_w(b)
        return carry

    lax.fori_loop(0, G_NGRP // 2 - 1, pair, 0)
    g = G_NGRP - 2
    fire_g(g + 1, 1)
    drain_g(0)
    fire_w(g, 0)
    drain_w(0)
    drain_g(1)
    fire_w(g + 1, 1)
    drain_w(1)

    toff = G_NGRP * G_GROUP
    pltpu.async_copy(
        table.at[idx_v.at[pl.ds(toff, G_TAIL)]],
        bufa[0].at[pl.ds(0, G_TAIL)], gsa).wait()
    pltpu.sync_copy(bufa[0].at[pl.ds(0, G_TAIL)],
                    out.at[pl.ds(base + toff, G_TAIL)])


S_NCHUNK = EH // S_CH
S_BASE = S_NCHUNK // NS
S_REM = S_NCHUNK - S_BASE * NS

@functools.partial(
    pl.kernel,
    out_type=jax.ShapeDtypeStruct((NC * N_NODES, D), jnp.float32),
    mesh=_mesh,
    scratch_types=[
        [pltpu.VMEM((S_CH,), jnp.int32)] * 2,
        [pltpu.VMEM((S_CH, D), jnp.float32)] * 2,
        pltpu.VMEM_SHARED((N_NODES, D), jnp.float32),
        pltpu.SemaphoreType.DMA,
        pltpu.SemaphoreType.DMA,
        pltpu.SemaphoreType.DMA,
        pltpu.SemaphoreType.DMA,
    ],
)
def _scatter(he0, he1, dst0, dst1, zeros_hbm, out, idx_c, rv, acc, is0, is1,
             ls0, ls1):
    c = lax.axis_index("c")
    s = lax.axis_index("s")
    isem = (is0, is1)
    lsem = (ls0, ls1)

    zb = pl.multiple_of(s * ZR, 8)
    pltpu.sync_copy(zeros_hbm.at[pl.ds(zb, ZR)], acc.at[pl.ds(zb, ZR)])

    @pl.when(s == NS - 1)
    def _():
        tb = pl.multiple_of(NS * ZR, 8)
        pltpu.sync_copy(zeros_hbm.at[pl.ds(tb, ZTAIL)],
                        acc.at[pl.ds(tb, ZTAIL)])

    plsc.subcore_barrier()

    def half_loop(rows_hbm, dst1h):
        nch = S_BASE + jnp.where(s < S_REM, 1, 0)
        cbase = s * S_BASE + jnp.minimum(s, S_REM)

        def fire(j, par):
            goff = pl.multiple_of((cbase + j) * S_CH, S_CH)
            pltpu.async_copy(dst1h.at[pl.ds(goff, S_CH)], idx_c[par],
                             isem[par])
            pltpu.async_copy(rows_hbm.at[pl.ds(goff, S_CH)], rv[par],
                             lsem[par])

        def wait(par):
            pltpu.make_async_copy(dst1h.at[pl.ds(0, S_CH)], idx_c[par],
                                  isem[par]).wait()
            pltpu.make_async_copy(rows_hbm.at[pl.ds(0, S_CH)], rv[par],
                                  lsem[par]).wait()

        def add(par):
            pltpu.sync_copy(rv[par], acc.at[idx_c[par]], add=True)

        fire(0, 0)

        def pair(p, carry):
            j = 2 * p
            for b in range(2):
                wait(b)
                fire(j + b + 1, 1 - b)
                add(b)
            return carry

        lax.fori_loop(0, (S_BASE - 2) // 2, pair, 0)
        wait(0)
        fire(S_BASE - 1, 1)
        add(0)
        wait(1)

        @pl.when(s < S_REM)
        def _():
            fire(S_BASE, 0)

        add(1)

        @pl.when(s < S_REM)
        def _():
            wait(0)
            add(0)

    @pl.when(c == 0)
    def _():
        half_loop(he0, dst0)

    @pl.when(c == 1)
    def _():
        half_loop(he1, dst1)

    plsc.subcore_barrier()
    zo = pl.multiple_of(c * N_NODES + s * ZR, 8)
    pltpu.sync_copy(acc.at[pl.ds(zb, ZR)], out.at[pl.ds(zo, ZR)])

    @pl.when(s == NS - 1)
    def _():
        tb = pl.multiple_of(NS * ZR, 8)
        to = pl.multiple_of(c * N_NODES + NS * ZR, 8)
        pltpu.sync_copy(acc.at[pl.ds(tb, ZTAIL)], out.at[pl.ds(to, ZTAIL)])


BE = 4000
BN = 2000
_PREC = lax.Precision.DEFAULT


def _full(shape):
    return pl.BlockSpec(shape, lambda i: tuple(0 for _ in shape))


def _edge_body(g1, g2, he, we, we2, b1, b2, gg, bb, out):
    hev = he[...]
    x = g1[...] + g2[...] + b1[...] + jnp.dot(
        hev, we[...], preferred_element_type=jnp.float32, precision=_PREC)
    h = jnp.maximum(x, 0.0)
    e = b2[...] + jnp.dot(
        h, we2[...], preferred_element_type=jnp.float32, precision=_PREC)
    m = jnp.mean(e, axis=-1, keepdims=True)
    v = jnp.mean((e - m) ** 2, axis=-1, keepdims=True)
    e = (e - m) * lax.rsqrt(v + 1e-5) * gg[...] + bb[...]
    out[...] = hev + e


def _edge_call(G, he, we, we2, b1, b2, gg, bb):
    nblk = EH // BE
    return pl.pallas_call(
        _edge_body,
        grid=(nblk,),
        in_specs=[
            pl.BlockSpec((BE, D), lambda i: (i, 0)),
            pl.BlockSpec((BE, D), lambda i: (nblk + i, 0)),
            pl.BlockSpec((BE, D), lambda i: (i, 0)),
            _full((D, D)), _full((D, D)),
            _full((1, D)), _full((1, D)), _full((1, D)), _full((1, D)),
        ],
        out_specs=pl.BlockSpec((BE, D), lambda i: (i, 0)),
        out_shape=jax.ShapeDtypeStruct((EH, D), jnp.float32),
        compiler_params=pltpu.CompilerParams(
            dimension_semantics=("arbitrary",)),
    )(G, G, he, we, we2, b1, b2, gg, bb)


def _node_core(hv, agg, w1a, w1b, b1, w2, b2, gg, bb):
    x = b1[...] + jnp.dot(
        hv, w1a[...], preferred_element_type=jnp.float32, precision=_PREC)
    x = x + jnp.dot(
        agg, w1b[...], preferred_element_type=jnp.float32, precision=_PREC)
    h = jnp.maximum(x, 0.0)
    n = b2[...] + jnp.dot(
        h, w2[...], preferred_element_type=jnp.float32, precision=_PREC)
    m = jnp.mean(n, axis=-1, keepdims=True)
    v = jnp.mean((n - m) ** 2, axis=-1, keepdims=True)
    n = (n - m) * lax.rsqrt(v + 1e-5) * gg[...] + bb[...]
    return hv + n


def _node_body_p(hn, ga, gb, w1a, w1b, b1, w2, b2, gg, bb, wnx, hout, pout):
    hv = hn[...]
    hnew = _node_core(hv, ga[...] + gb[...], w1a, w1b, b1, w2, b2, gg, bb)
    hout[...] = hnew
    pout[0] = jnp.dot(hnew, wnx[0],
                      preferred_element_type=jnp.float32, precision=_PREC)
    pout[1] = jnp.dot(hnew, wnx[1],
                      preferred_element_type=jnp.float32, precision=_PREC)


def _node_body(hn, ga, gb, w1a, w1b, b1, w2, b2, gg, bb, hout):
    hv = hn[...]
    hout[...] = _node_core(hv, ga[...] + gb[...], w1a, w1b, b1, w2, b2, gg, bb)


def _node_specs():
    nblk = N_NODES // BN
    return [
        pl.BlockSpec((BN, D), lambda i: (i, 0)),
        pl.BlockSpec((BN, D), lambda i: (i, 0)),
        pl.BlockSpec((BN, D), lambda i: (nblk + i, 0)),
        _full((D, D)), _full((D, D)), _full((1, D)),
        _full((D, D)), _full((1, D)), _full((1, D)), _full((1, D)),
    ]


def _node_call_p(hn, agg2, w1a, w1b, b1, w2, b2, gg, bb, wnx):
    return pl.pallas_call(
        _node_body_p,
        grid=(N_NODES // BN,),
        in_specs=_node_specs() + [_full((2, D, D))],
        out_specs=[
            pl.BlockSpec((BN, D), lambda i: (i, 0)),
            pl.BlockSpec((2, BN, D), lambda i: (0, i, 0)),
        ],
        out_shape=[
            jax.ShapeDtypeStruct((N_NODES, D), jnp.float32),
            jax.ShapeDtypeStruct((2, N_NODES, D), jnp.float32),
        ],
        compiler_params=pltpu.CompilerParams(
            dimension_semantics=("arbitrary",)),
    )(hn, agg2, agg2, w1a, w1b, b1, w2, b2, gg, bb, wnx)


def _node_call(hn, agg2, w1a, w1b, b1, w2, b2, gg, bb):
    return pl.pallas_call(
        _node_body,
        grid=(N_NODES // BN,),
        in_specs=_node_specs(),
        out_specs=pl.BlockSpec((BN, D), lambda i: (i, 0)),
        out_shape=jax.ShapeDtypeStruct((N_NODES, D), jnp.float32),
        compiler_params=pltpu.CompilerParams(
            dimension_semantics=("arbitrary",)),
    )(hn, agg2, agg2, w1a, w1b, b1, w2, b2, gg, bb)


def _p0_body(hn, wnx, pout):
    hv = hn[...]
    pout[0] = jnp.dot(hv, wnx[0],
                      preferred_element_type=jnp.float32, precision=_PREC)
    pout[1] = jnp.dot(hv, wnx[1],
                      preferred_element_type=jnp.float32, precision=_PREC)


def _p0_call(hn, wnx):
    return pl.pallas_call(
        _p0_body,
        grid=(N_NODES // BN,),
        in_specs=[pl.BlockSpec((BN, D), lambda i: (i, 0)), _full((2, D, D))],
        out_specs=pl.BlockSpec((2, BN, D), lambda i: (0, i, 0)),
        out_shape=jax.ShapeDtypeStruct((2, N_NODES, D), jnp.float32),
        compiler_params=pltpu.CompilerParams(
            dimension_semantics=("arbitrary",)),
    )(hn, wnx)


def kernel(h_node, h_edge, edge_index, We1, be1, We2, be2, ge, bbe,
           Wn1, bn1, Wn2, bn2, gn, bbn):
    src = edge_index[0].astype(jnp.int32)
    dst = edge_index[1].astype(jnp.int32)
    dst_h = (dst[:EH], dst[EH:])
    idx_h = (jnp.concatenate([src[:EH], dst_h[0] + N_NODES]),
             jnp.concatenate([src[EH:], dst_h[1] + N_NODES]))
    he = [h_edge[:EH], h_edge[EH:]]
    zeros = jnp.zeros((N_NODES, D), jnp.float32)

    P = _p0_call(h_node, We1[0, :2 * D].reshape(2, D, D))
    for l in range(NUM_CONVS):
        Pflat = P.reshape(2 * N_NODES, D)
        G = [_gather(Pflat, idx_h[h]) for h in range(2)]
        for h in range(2):
            he[h] = _edge_call(G[h], he[h], We1[l, 2 * D:], We2[l],
                               be1[l][None], be2[l][None],
                               ge[l][None], bbe[l][None])
        agg2 = _scatter(he[0], he[1], dst_h[0], dst_h[1], zeros)
        args = (h_node, agg2, Wn1[l, :D], Wn1[l, D:], bn1[l][None],
                Wn2[l], bn2[l][None], gn[l][None], bbn[l][None])
        if l + 1 < NUM_CONVS:
            h_node, P = _node_call_p(*args, We1[l + 1, :2 * D].reshape(2, D, D))
        else:
            h_node = _node_call(*args)
    return (h_node, jnp.concatenate(he))


# speedup vs baseline: 1.0377x; 1.0377x over previous
import functools

import jax
import jax.numpy as jnp
from jax import lax
from jax.experimental import pallas as pl
from jax.experimental.pallas import tpu as pltpu
from jax.experimental.pallas import tpu_sc as plsc

NUM_CONVS = 4
D = 128
N_NODES = 10000
N_EDGES = 320000
EH = N_EDGES // 2
NC, NS = 2, 16
NW = NC * NS

G_CH = 96
G_K = 4
S_CH = 128
ZR = 624
ZTAIL = N_NODES - NS * ZR

_mesh = plsc.VectorSubcoreMesh(
    core_axis_name="c", subcore_axis_name="s", num_cores=NC, num_subcores=NS)


GPW = 2 * EH // NW
G_GROUP = G_CH * G_K
G_NGRP = GPW // G_GROUP
G_TAIL = GPW - G_NGRP * G_GROUP

@functools.partial(
    pl.kernel,
    out_type=jax.ShapeDtypeStruct((2 * EH, D), jnp.float32),
    mesh=_mesh,
    scratch_types=[
        pltpu.VMEM((GPW,), jnp.int32),
        [pltpu.VMEM((G_CH, D), jnp.float32)] * G_K,
        [pltpu.VMEM((G_CH, D), jnp.float32)] * G_K,
        pltpu.SemaphoreType.DMA,
        pltpu.SemaphoreType.DMA,
        pltpu.SemaphoreType.DMA,
        pltpu.SemaphoreType.DMA,
    ],
)
def _gather(table, idx, out, idx_v, bufa, bufb, gsa, gsb, wsa, wsb):
    bufs = (bufa, bufb)
    gs = (gsa, gsb)
    ws = (wsa, wsb)
    wid = lax.axis_index("s") * NC + lax.axis_index("c")
    base = pl.multiple_of(wid * GPW, GPW)
    pltpu.sync_copy(idx.at[pl.ds(base, GPW)], idx_v)

    def fire_g(g, par):
        goff = pl.multiple_of(g * G_GROUP, G_GROUP)
        for k in range(G_K):
            pltpu.async_copy(
                table.at[idx_v.at[pl.ds(goff + k * G_CH, G_CH)]],
                bufs[par][k], gs[par])

    def drain_g(par):
        for k in range(G_K):
            pltpu.make_async_copy(
                table.at[idx_v.at[pl.ds(0, G_CH)]], bufs[par][k],
                gs[par]).wait()

    def fire_w(g, par):
        goff = pl.multiple_of(g * G_GROUP, G_GROUP)
        for k in range(G_K):
            pltpu.async_copy(
                bufs[par][k],
                out.at[pl.ds(base + goff + k * G_CH, G_CH)], ws[par])

    def drain_w(par):
        for k in range(G_K):
            pltpu.make_async_copy(
                bufs[par][k], out.at[pl.ds(base, G_CH)], ws[par]).wait()

    fire_g(0, 0)

    def pair(p, carry):
        g = 2 * p
        for b in range(2):
            fire_g(g + b + 1, 1 - b)
            drain_g(b)
            fire_w(g + b, b)
            drain_w(b)
        return carry

    lax.fori_loop(0, G_NGRP // 2 - 1, pair, 0)
    g = G_NGRP - 2
    fire_g(g + 1, 1)
    drain_g(0)
    fire_w(g, 0)
    drain_w(0)
    drain_g(1)
    fire_w(g + 1, 1)
    drain_w(1)

    toff = G_NGRP * G_GROUP
    pltpu.async_copy(
        table.at[idx_v.at[pl.ds(toff, G_TAIL)]],
        bufa[0].at[pl.ds(0, G_TAIL)], gsa).wait()
    pltpu.sync_copy(bufa[0].at[pl.ds(0, G_TAIL)],
                    out.at[pl.ds(base + toff, G_TAIL)])


S_NCHUNK = EH // S_CH
S_BASE = S_NCHUNK // NW
S_REM = S_NCHUNK - S_BASE * NW

@functools.partial(
    pl.kernel,
    out_type=jax.ShapeDtypeStruct((NC * N_NODES, D), jnp.float32),
    mesh=_mesh,
    scratch_types=[
        [pltpu.VMEM((S_CH,), jnp.int32)] * 2,
        [pltpu.VMEM((S_CH, D), jnp.float32)] * 2,
        pltpu.VMEM_SHARED((N_NODES, D), jnp.float32),
        pltpu.SemaphoreType.DMA,
        pltpu.SemaphoreType.DMA,
        pltpu.SemaphoreType.DMA,
        pltpu.SemaphoreType.DMA,
    ],
)
def _scatter(rows_hbm, dst1, zeros_hbm, out, idx_c, rv, acc, is0, is1,
             ls0, ls1):
    c = lax.axis_index("c")
    s = lax.axis_index("s")
    wid = s * NC + c
    isem = (is0, is1)
    lsem = (ls0, ls1)

    nch = S_BASE + jnp.where(wid < S_REM, 1, 0)
    cbase = wid * S_BASE + jnp.minimum(wid, S_REM)

    def fire(j, par):
        goff = pl.multiple_of((cbase + j) * S_CH, S_CH)
        pltpu.async_copy(dst1.at[pl.ds(goff, S_CH)], idx_c[par], isem[par])
        pltpu.async_copy(rows_hbm.at[pl.ds(goff, S_CH)], rv[par], lsem[par])

    def wait(par):
        pltpu.make_async_copy(dst1.at[pl.ds(0, S_CH)], idx_c[par],
                              isem[par]).wait()
        pltpu.make_async_copy(rows_hbm.at[pl.ds(0, S_CH)], rv[par],
                              lsem[par]).wait()

    def add(par):
        pltpu.sync_copy(rv[par], acc.at[idx_c[par]], add=True)

    fire(0, 0)
    zb = pl.multiple_of(s * ZR, 8)
    pltpu.sync_copy(zeros_hbm.at[pl.ds(zb, ZR)], acc.at[pl.ds(zb, ZR)])

    @pl.when(s == NS - 1)
    def _():
        tb = pl.multiple_of(NS * ZR, 8)
        pltpu.sync_copy(zeros_hbm.at[pl.ds(tb, ZTAIL)],
                        acc.at[pl.ds(tb, ZTAIL)])

    plsc.subcore_barrier()

    def pair(p, carry):
        j = 2 * p
        for b in range(2):
            wait(b)
            fire(j + b + 1, 1 - b)
            add(b)
        return carry

    lax.fori_loop(0, (S_BASE - 1) // 2, pair, 0)
    wait(0)

    @pl.when(wid < S_REM)
    def _():
        fire(S_BASE, 1)

    add(0)

    @pl.when(wid < S_REM)
    def _():
        wait(1)
        add(1)

    plsc.subcore_barrier()
    zo = pl.multiple_of(c * N_NODES + s * ZR, 8)
    pltpu.sync_copy(acc.at[pl.ds(zb, ZR)], out.at[pl.ds(zo, ZR)])

    @pl.when(s == NS - 1)
    def _():
        tb = pl.multiple_of(NS * ZR, 8)
        to = pl.multiple_of(c * N_NODES + NS * ZR, 8)
        pltpu.sync_copy(acc.at[pl.ds(tb, ZTAIL)], out.at[pl.ds(to, ZTAIL)])


BE = 4000
BN = 2000
_PREC = lax.Precision.DEFAULT


def _full(shape):
    return pl.BlockSpec(shape, lambda i: tuple(0 for _ in shape))


def _edge_body(g1, g2, he, we, we2, b1, b2, gg, bb, out):
    hev = he[...]
    x = g1[...] + g2[...] + b1[...] + jnp.dot(
        hev, we[...], preferred_element_type=jnp.float32, precision=_PREC)
    h = jnp.maximum(x, 0.0)
    e = b2[...] + jnp.dot(
        h, we2[...], preferred_element_type=jnp.float32, precision=_PREC)
    m = jnp.mean(e, axis=-1, keepdims=True)
    v = jnp.mean((e - m) ** 2, axis=-1, keepdims=True)
    e = (e - m) * lax.rsqrt(v + 1e-5) * gg[...] + bb[...]
    out[...] = hev + e


def _edge_call(G, he, we, we2, b1, b2, gg, bb):
    nblk = EH // BE
    return pl.pallas_call(
        _edge_body,
        grid=(nblk,),
        in_specs=[
            pl.BlockSpec((BE, D), lambda i: (i, 0)),
            pl.BlockSpec((BE, D), lambda i: (nblk + i, 0)),
            pl.BlockSpec((BE, D), lambda i: (i, 0)),
            _full((D, D)), _full((D, D)),
            _full((1, D)), _full((1, D)), _full((1, D)), _full((1, D)),
        ],
        out_specs=pl.BlockSpec((BE, D), lambda i: (i, 0)),
        out_shape=jax.ShapeDtypeStruct((EH, D), jnp.float32),
        compiler_params=pltpu.CompilerParams(
            dimension_semantics=("arbitrary",)),
    )(G, G, he, we, we2, b1, b2, gg, bb)


def _node_core(hv, agg, w1a, w1b, b1, w2, b2, gg, bb):
    x = b1[...] + jnp.dot(
        hv, w1a[...], preferred_element_type=jnp.float32, precision=_PREC)
    x = x + jnp.dot(
        agg, w1b[...], preferred_element_type=jnp.float32, precision=_PREC)
    h = jnp.maximum(x, 0.0)
    n = b2[...] + jnp.dot(
        h, w2[...], preferred_element_type=jnp.float32, precision=_PREC)
    m = jnp.mean(n, axis=-1, keepdims=True)
    v = jnp.mean((n - m) ** 2, axis=-1, keepdims=True)
    n = (n - m) * lax.rsqrt(v + 1e-5) * gg[...] + bb[...]
    return hv + n


def _agg4(a0a, a0b, a1a, a1b):
    return (a0a[...] + a0b[...]) + (a1a[...] + a1b[...])


def _node_body_p(hn, a0a, a0b, a1a, a1b, w1a, w1b, b1, w2, b2, gg, bb, wnx,
                 hout, pout):
    hv = hn[...]
    hnew = _node_core(hv, _agg4(a0a, a0b, a1a, a1b),
                      w1a, w1b, b1, w2, b2, gg, bb)
    hout[...] = hnew
    pout[0] = jnp.dot(hnew, wnx[0],
                      preferred_element_type=jnp.float32, precision=_PREC)
    pout[1] = jnp.dot(hnew, wnx[1],
                      preferred_element_type=jnp.float32, precision=_PREC)


def _node_body(hn, a0a, a0b, a1a, a1b, w1a, w1b, b1, w2, b2, gg, bb, hout):
    hv = hn[...]
    hout[...] = _node_core(hv, _agg4(a0a, a0b, a1a, a1b),
                           w1a, w1b, b1, w2, b2, gg, bb)


def _node_specs():
    nblk = N_NODES // BN
    return [
        pl.BlockSpec((BN, D), lambda i: (i, 0)),
        pl.BlockSpec((BN, D), lambda i: (i, 0)),
        pl.BlockSpec((BN, D), lambda i: (nblk + i, 0)),
        pl.BlockSpec((BN, D), lambda i: (i, 0)),
        pl.BlockSpec((BN, D), lambda i: (nblk + i, 0)),
        _full((D, D)), _full((D, D)), _full((1, D)),
        _full((D, D)), _full((1, D)), _full((1, D)), _full((1, D)),
    ]


def _node_call_p(hn, a0, a1, w1a, w1b, b1, w2, b2, gg, bb, wnx):
    return pl.pallas_call(
        _node_body_p,
        grid=(N_NODES // BN,),
        in_specs=_node_specs() + [_full((2, D, D))],
        out_specs=[
            pl.BlockSpec((BN, D), lambda i: (i, 0)),
            pl.BlockSpec((2, BN, D), lambda i: (0, i, 0)),
        ],
        out_shape=[
            jax.ShapeDtypeStruct((N_NODES, D), jnp.float32),
            jax.ShapeDtypeStruct((2, N_NODES, D), jnp.float32),
        ],
        compiler_params=pltpu.CompilerParams(
            dimension_semantics=("arbitrary",)),
    )(hn, a0, a0, a1, a1, w1a, w1b, b1, w2, b2, gg, bb, wnx)


def _node_call(hn, a0, a1, w1a, w1b, b1, w2, b2, gg, bb):
    return pl.pallas_call(
        _node_body,
        grid=(N_NODES // BN,),
        in_specs=_node_specs(),
        out_specs=pl.BlockSpec((BN, D), lambda i: (i, 0)),
        out_shape=jax.ShapeDtypeStruct((N_NODES, D), jnp.float32),
        compiler_params=pltpu.CompilerParams(
            dimension_semantics=("arbitrary",)),
    )(hn, a0, a0, a1, a1, w1a, w1b, b1, w2, b2, gg, bb)


def _p0_body(hn, wnx, pout):
    hv = hn[...]
    pout[0] = jnp.dot(hv, wnx[0],
                      preferred_element_type=jnp.float32, precision=_PREC)
    pout[1] = jnp.dot(hv, wnx[1],
                      preferred_element_type=jnp.float32, precision=_PREC)


def _p0_call(hn, wnx):
    return pl.pallas_call(
        _p0_body,
        grid=(N_NODES // BN,),
        in_specs=[pl.BlockSpec((BN, D), lambda i: (i, 0)), _full((2, D, D))],
        out_specs=pl.BlockSpec((2, BN, D), lambda i: (0, i, 0)),
        out_shape=jax.ShapeDtypeStruct((2, N_NODES, D), jnp.float32),
        compiler_params=pltpu.CompilerParams(
            dimension_semantics=("arbitrary",)),
    )(hn, wnx)


def kernel(h_node, h_edge, edge_index, We1, be1, We2, be2, ge, bbe,
           Wn1, bn1, Wn2, bn2, gn, bbn):
    src = edge_index[0].astype(jnp.int32)
    dst = edge_index[1].astype(jnp.int32)
    dst_h = (dst[:EH], dst[EH:])
    idx_h = (jnp.concatenate([src[:EH], dst_h[0] + N_NODES]),
             jnp.concatenate([src[EH:], dst_h[1] + N_NODES]))
    he = [h_edge[:EH], h_edge[EH:]]
    zeros = jnp.zeros((N_NODES, D), jnp.float32)

    P = _p0_call(h_node, We1[0, :2 * D].reshape(2, D, D))
    for l in range(NUM_CONVS):
        Pflat = P.reshape(2 * N_NODES, D)
        G = [_gather(Pflat, idx_h[h]) for h in range(2)]
        agg = [None, None]
        for h in range(2):
            he[h] = _edge_call(G[h], he[h], We1[l, 2 * D:], We2[l],
                               be1[l][None], be2[l][None],
                               ge[l][None], bbe[l][None])
            agg[h] = _scatter(he[h], dst_h[h], zeros)
        args = (h_node, agg[0], agg[1], Wn1[l, :D], Wn1[l, D:], bn1[l][None],
                Wn2[l], bn2[l][None], gn[l][None], bbn[l][None])
        if l + 1 < NUM_CONVS:
            h_node, P = _node_call_p(*args, We1[l + 1, :2 * D].reshape(2, D, D))
        else:
            h_node = _node_call(*args)
    return (h_node, jnp.concatenate(he))


# speedup vs baseline: 1.0381x; 1.0004x over previous
import functools

import jax
import jax.numpy as jnp
from jax import lax
from jax.experimental import pallas as pl
from jax.experimental.pallas import tpu as pltpu
from jax.experimental.pallas import tpu_sc as plsc

NUM_CONVS = 4
D = 128
N_NODES = 10000
N_EDGES = 320000
EH = N_EDGES // 2
NC, NS = 2, 16
NW = NC * NS

G_CH = 128
G_K = 3
S_CH = 128
ZR = 624
ZTAIL = N_NODES - NS * ZR

_mesh = plsc.VectorSubcoreMesh(
    core_axis_name="c", subcore_axis_name="s", num_cores=NC, num_subcores=NS)


GPW = 2 * EH // NW
G_GROUP = G_CH * G_K
G_NGRP = GPW // G_GROUP
G_TAIL = GPW - G_NGRP * G_GROUP

@functools.partial(
    pl.kernel,
    out_type=jax.ShapeDtypeStruct((2 * EH, D), jnp.float32),
    mesh=_mesh,
    scratch_types=[
        pltpu.VMEM((GPW,), jnp.int32),
        [pltpu.VMEM((G_CH, D), jnp.float32)] * G_K,
        [pltpu.VMEM((G_CH, D), jnp.float32)] * G_K,
        pltpu.SemaphoreType.DMA,
        pltpu.SemaphoreType.DMA,
        pltpu.SemaphoreType.DMA,
        pltpu.SemaphoreType.DMA,
    ],
)
def _gather(table, idx, out, idx_v, bufa, bufb, gsa, gsb, wsa, wsb):
    bufs = (bufa, bufb)
    gs = (gsa, gsb)
    ws = (wsa, wsb)
    wid = lax.axis_index("s") * NC + lax.axis_index("c")
    base = pl.multiple_of(wid * GPW, GPW)
    pltpu.sync_copy(idx.at[pl.ds(base, GPW)], idx_v)

    def fire_g(g, par):
        goff = pl.multiple_of(g * G_GROUP, G_GROUP)
        for k in range(G_K):
            pltpu.async_copy(
                table.at[idx_v.at[pl.ds(goff + k * G_CH, G_CH)]],
                bufs[par][k], gs[par])

    def drain_g(par):
        for k in range(G_K):
            pltpu.make_async_copy(
                table.at[idx_v.at[pl.ds(0, G_CH)]], bufs[par][k],
                gs[par]).wait()

    def fire_w(g, par):
        goff = pl.multiple_of(g * G_GROUP, G_GROUP)
        for k in range(G_K):
            pltpu.async_copy(
                bufs[par][k],
                out.at[pl.ds(base + goff + k * G_CH, G_CH)], ws[par])

    def drain_w(par):
        for k in range(G_K):
            pltpu.make_async_copy(
                bufs[par][k], out.at[pl.ds(base, G_CH)], ws[par]).wait()

    fire_g(0, 0)

    def pair(p, carry):
        g = 2 * p
        for b in range(2):
            fire_g(g + b + 1, 1 - b)
            drain_g(b)
            fire_w(g + b, b)
            drain_w(b)
        return carry

    lax.fori_loop(0, G_NGRP // 2 - 1, pair, 0)
    g = G_NGRP - 2
    fire_g(g + 1, 1)
    drain_g(0)
    fire_w(g, 0)
    drain_w(0)
    drain_g(1)
    fire_w(g + 1, 1)
    drain_w(1)

    toff = G_NGRP * G_GROUP
    pltpu.async_copy(
        table.at[idx_v.at[pl.ds(toff, G_TAIL)]],
        bufa[0].at[pl.ds(0, G_TAIL)], gsa).wait()
    pltpu.sync_copy(bufa[0].at[pl.ds(0, G_TAIL)],
                    out.at[pl.ds(base + toff, G_TAIL)])


S_NCHUNK = EH // S_CH
S_BASE = S_NCHUNK // NW
S_REM = S_NCHUNK - S_BASE * NW

@functools.partial(
    pl.kernel,
    out_type=jax.ShapeDtypeStruct((NC * N_NODES, D), jnp.float32),
    mesh=_mesh,
    scratch_types=[
        [pltpu.VMEM((S_CH,), jnp.int32)] * 2,
        [pltpu.VMEM((S_CH, D), jnp.float32)] * 2,
        pltpu.VMEM_SHARED((N_NODES, D), jnp.float32),
        pltpu.SemaphoreType.DMA,
        pltpu.SemaphoreType.DMA,
        pltpu.SemaphoreType.DMA,
        pltpu.SemaphoreType.DMA,
    ],
)
def _scatter(rows_hbm, dst1, zeros_hbm, out, idx_c, rv, acc, is0, is1,
             ls0, ls1):
    c = lax.axis_index("c")
    s = lax.axis_index("s")
    wid = s * NC + c
    isem = (is0, is1)
    lsem = (ls0, ls1)

    nch = S_BASE + jnp.where(wid < S_REM, 1, 0)
    cbase = wid * S_BASE + jnp.minimum(wid, S_REM)

    def fire(j, par):
        goff = pl.multiple_of((cbase + j) * S_CH, S_CH)
        pltpu.async_copy(dst1.at[pl.ds(goff, S_CH)], idx_c[par], isem[par])
        pltpu.async_copy(rows_hbm.at[pl.ds(goff, S_CH)], rv[par], lsem[par])

    def wait(par):
        pltpu.make_async_copy(dst1.at[pl.ds(0, S_CH)], idx_c[par],
                              isem[par]).wait()
        pltpu.make_async_copy(rows_hbm.at[pl.ds(0, S_CH)], rv[par],
                              lsem[par]).wait()

    def add(par):
        pltpu.sync_copy(rv[par], acc.at[idx_c[par]], add=True)

    fire(0, 0)
    zb = pl.multiple_of(s * ZR, 8)
    pltpu.sync_copy(zeros_hbm.at[pl.ds(zb, ZR)], acc.at[pl.ds(zb, ZR)])

    @pl.when(s == NS - 1)
    def _():
        tb = pl.multiple_of(NS * ZR, 8)
        pltpu.sync_copy(zeros_hbm.at[pl.ds(tb, ZTAIL)],
                        acc.at[pl.ds(tb, ZTAIL)])

    plsc.subcore_barrier()

    def pair(p, carry):
        j = 2 * p
        for b in range(2):
            wait(b)
            fire(j + b + 1, 1 - b)
            add(b)
        return carry

    lax.fori_loop(0, (S_BASE - 1) // 2, pair, 0)
    wait(0)

    @pl.when(wid < S_REM)
    def _():
        fire(S_BASE, 1)

    add(0)

    @pl.when(wid < S_REM)
    def _():
        wait(1)
        add(1)

    plsc.subcore_barrier()
    zo = pl.multiple_of(c * N_NODES + s * ZR, 8)
    pltpu.sync_copy(acc.at[pl.ds(zb, ZR)], out.at[pl.ds(zo, ZR)])

    @pl.when(s == NS - 1)
    def _():
        tb = pl.multiple_of(NS * ZR, 8)
        to = pl.multiple_of(c * N_NODES + NS * ZR, 8)
        pltpu.sync_copy(acc.at[pl.ds(tb, ZTAIL)], out.at[pl.ds(to, ZTAIL)])


BE = 4000
BN = 2000
_PREC = lax.Precision.DEFAULT


def _full(shape):
    return pl.BlockSpec(shape, lambda i: tuple(0 for _ in shape))


def _edge_body(g1, g2, he, we, we2, b1, b2, gg, bb, out):
    hev = he[...]
    x = g1[...] + g2[...] + b1[...] + jnp.dot(
        hev, we[...], preferred_element_type=jnp.float32, precision=_PREC)
    h = jnp.maximum(x, 0.0)
    e = b2[...] + jnp.dot(
        h, we2[...], preferred_element_type=jnp.float32, precision=_PREC)
    m = jnp.mean(e, axis=-1, keepdims=True)
    v = jnp.mean((e - m) ** 2, axis=-1, keepdims=True)
    e = (e - m) * lax.rsqrt(v + 1e-5) * gg[...] + bb[...]
    out[...] = hev + e


def _edge_call(G, he, we, we2, b1, b2, gg, bb):
    nblk = EH // BE
    return pl.pallas_call(
        _edge_body,
        grid=(nblk,),
        in_specs=[
            pl.BlockSpec((BE, D), lambda i: (i, 0)),
            pl.BlockSpec((BE, D), lambda i: (nblk + i, 0)),
            pl.BlockSpec((BE, D), lambda i: (i, 0)),
            _full((D, D)), _full((D, D)),
            _full((1, D)), _full((1, D)), _full((1, D)), _full((1, D)),
        ],
        out_specs=pl.BlockSpec((BE, D), lambda i: (i, 0)),
        out_shape=jax.ShapeDtypeStruct((EH, D), jnp.float32),
        compiler_params=pltpu.CompilerParams(
            dimension_semantics=("arbitrary",)),
    )(G, G, he, we, we2, b1, b2, gg, bb)


def _node_core(hv, agg, w1a, w1b, b1, w2, b2, gg, bb):
    x = b1[...] + jnp.dot(
        hv, w1a[...], preferred_element_type=jnp.float32, precision=_PREC)
    x = x + jnp.dot(
        agg, w1b[...], preferred_element_type=jnp.float32, precision=_PREC)
    h = jnp.maximum(x, 0.0)
    n = b2[...] + jnp.dot(
        h, w2[...], preferred_element_type=jnp.float32, precision=_PREC)
    m = jnp.mean(n, axis=-1, keepdims=True)
    v = jnp.mean((n - m) ** 2, axis=-1, keepdims=True)
    n = (n - m) * lax.rsqrt(v + 1e-5) * gg[...] + bb[...]
    return hv + n


def _agg4(a0a, a0b, a1a, a1b):
    return (a0a[...] + a0b[...]) + (a1a[...] + a1b[...])


def _node_body_p(hn, a0a, a0b, a1a, a1b, w1a, w1b, b1, w2, b2, gg, bb, wnx,
                 hout, pout):
    hv = hn[...]
    hnew = _node_core(hv, _agg4(a0a, a0b, a1a, a1b),
                      w1a, w1b, b1, w2, b2, gg, bb)
    hout[...] = hnew
    pout[0] = jnp.dot(hnew, wnx[0],
                      preferred_element_type=jnp.float32, precision=_PREC)
    pout[1] = jnp.dot(hnew, wnx[1],
                      preferred_element_type=jnp.float32, precision=_PREC)


def _node_body(hn, a0a, a0b, a1a, a1b, w1a, w1b, b1, w2, b2, gg, bb, hout):
    hv = hn[...]
    hout[...] = _node_core(hv, _agg4(a0a, a0b, a1a, a1b),
                           w1a, w1b, b1, w2, b2, gg, bb)


def _node_specs():
    nblk = N_NODES // BN
    return [
        pl.BlockSpec((BN, D), lambda i: (i, 0)),
        pl.BlockSpec((BN, D), lambda i: (i, 0)),
        pl.BlockSpec((BN, D), lambda i: (nblk + i, 0)),
        pl.BlockSpec((BN, D), lambda i: (i, 0)),
        pl.BlockSpec((BN, D), lambda i: (nblk + i, 0)),
        _full((D, D)), _full((D, D)), _full((1, D)),
        _full((D, D)), _full((1, D)), _full((1, D)), _full((1, D)),
    ]


def _node_call_p(hn, a0, a1, w1a, w1b, b1, w2, b2, gg, bb, wnx):
    return pl.pallas_call(
        _node_body_p,
        grid=(N_NODES // BN,),
        in_specs=_node_specs() + [_full((2, D, D))],
        out_specs=[
            pl.BlockSpec((BN, D), lambda i: (i, 0)),
            pl.BlockSpec((2, BN, D), lambda i: (0, i, 0)),
        ],
        out_shape=[
            jax.ShapeDtypeStruct((N_NODES, D), jnp.float32),
            jax.ShapeDtypeStruct((2, N_NODES, D), jnp.float32),
        ],
        compiler_params=pltpu.CompilerParams(
            dimension_semantics=("arbitrary",)),
    )(hn, a0, a0, a1, a1, w1a, w1b, b1, w2, b2, gg, bb, wnx)


def _node_call(hn, a0, a1, w1a, w1b, b1, w2, b2, gg, bb):
    return pl.pallas_call(
        _node_body,
        grid=(N_NODES // BN,),
        in_specs=_node_specs(),
        out_specs=pl.BlockSpec((BN, D), lambda i: (i, 0)),
        out_shape=jax.ShapeDtypeStruct((N_NODES, D), jnp.float32),
        compiler_params=pltpu.CompilerParams(
            dimension_semantics=("arbitrary",)),
    )(hn, a0, a0, a1, a1, w1a, w1b, b1, w2, b2, gg, bb)


def _p0_body(hn, wnx, pout):
    hv = hn[...]
    pout[0] = jnp.dot(hv, wnx[0],
                      preferred_element_type=jnp.float32, precision=_PREC)
    pout[1] = jnp.dot(hv, wnx[1],
                      preferred_element_type=jnp.float32, precision=_PREC)


def _p0_call(hn, wnx):
    return pl.pallas_call(
        _p0_body,
        grid=(N_NODES // BN,),
        in_specs=[pl.BlockSpec((BN, D), lambda i: (i, 0)), _full((2, D, D))],
        out_specs=pl.BlockSpec((2, BN, D), lambda i: (0, i, 0)),
        out_shape=jax.ShapeDtypeStruct((2, N_NODES, D), jnp.float32),
        compiler_params=pltpu.CompilerParams(
            dimension_semantics=("arbitrary",)),
    )(hn, wnx)


def kernel(h_node, h_edge, edge_index, We1, be1, We2, be2, ge, bbe,
           Wn1, bn1, Wn2, bn2, gn, bbn):
    src = edge_index[0].astype(jnp.int32)
    dst = edge_index[1].astype(jnp.int32)
    dst_h = (dst[:EH], dst[EH:])
    idx_h = (jnp.concatenate([src[:EH], dst_h[0] + N_NODES]),
             jnp.concatenate([src[EH:], dst_h[1] + N_NODES]))
    he = [h_edge[:EH], h_edge[EH:]]
    zeros = jnp.zeros((N_NODES, D), jnp.float32)

    P = _p0_call(h_node, We1[0, :2 * D].reshape(2, D, D))
    for l in range(NUM_CONVS):
        Pflat = P.reshape(2 * N_NODES, D)
        G = [_gather(Pflat, idx_h[h]) for h in range(2)]
        agg = [None, None]
        for h in range(2):
            he[h] = _edge_call(G[h], he[h], We1[l, 2 * D:], We2[l],
                               be1[l][None], be2[l][None],
                               ge[l][None], bbe[l][None])
            agg[h] = _scatter(he[h], dst_h[h], zeros)
        args = (h_node, agg[0], agg[1], Wn1[l, :D], Wn1[l, D:], bn1[l][None],
                Wn2[l], bn2[l][None], gn[l][None], bbn[l][None])
        if l + 1 < NUM_CONVS:
            h_node, P = _node_call_p(*args, We1[l + 1, :2 * D].reshape(2, D, D))
        else:
            h_node = _node_call(*args)
    return (h_node, jnp.concatenate(he))


# speedup vs baseline: 1.0416x; 1.0034x over previous
import functools

import jax
import jax.numpy as jnp
from jax import lax
from jax.experimental import pallas as pl
from jax.experimental.pallas import tpu as pltpu
from jax.experimental.pallas import tpu_sc as plsc

NUM_CONVS = 4
D = 128
N_NODES = 10000
N_EDGES = 320000
EH = N_EDGES // 2
NC, NS = 2, 16
NW = NC * NS

G_CH = 128
G_K = 3
S_CH = 128
ZR = 624
ZTAIL = N_NODES - NS * ZR

_mesh = plsc.VectorSubcoreMesh(
    core_axis_name="c", subcore_axis_name="s", num_cores=NC, num_subcores=NS)


GPW = 2 * EH // NW
G_GROUP = G_CH * G_K
G_NGRP = GPW // G_GROUP
G_TAIL = GPW - G_NGRP * G_GROUP

@functools.partial(
    pl.kernel,
    out_type=jax.ShapeDtypeStruct((2 * EH, D), jnp.float32),
    mesh=_mesh,
    scratch_types=[
        pltpu.VMEM((GPW,), jnp.int32),
        [pltpu.VMEM((G_CH, D), jnp.float32)] * G_K,
        [pltpu.VMEM((G_CH, D), jnp.float32)] * G_K,
        pltpu.SemaphoreType.DMA,
        pltpu.SemaphoreType.DMA,
        pltpu.SemaphoreType.DMA,
        pltpu.SemaphoreType.DMA,
    ],
)
def _gather(table, idx, out, idx_v, bufa, bufb, gsa, gsb, wsa, wsb):
    bufs = (bufa, bufb)
    gs = (gsa, gsb)
    ws = (wsa, wsb)
    wid = lax.axis_index("s") * NC + lax.axis_index("c")
    base = pl.multiple_of(wid * GPW, GPW)
    pltpu.sync_copy(idx.at[pl.ds(base, GPW)], idx_v)

    def fire_g(g, par):
        goff = pl.multiple_of(g * G_GROUP, G_GROUP)
        for k in range(G_K):
            pltpu.async_copy(
                table.at[idx_v.at[pl.ds(goff + k * G_CH, G_CH)]],
                bufs[par][k], gs[par])

    def drain_g(par):
        for k in range(G_K):
            pltpu.make_async_copy(
                table.at[idx_v.at[pl.ds(0, G_CH)]], bufs[par][k],
                gs[par]).wait()

    def fire_w(g, par):
        goff = pl.multiple_of(g * G_GROUP, G_GROUP)
        for k in range(G_K):
            pltpu.async_copy(
                bufs[par][k],
                out.at[pl.ds(base + goff + k * G_CH, G_CH)], ws[par])

    def drain_w(par):
        for k in range(G_K):
            pltpu.make_async_copy(
                bufs[par][k], out.at[pl.ds(base, G_CH)], ws[par]).wait()

    fire_g(0, 0)

    def pair(p, carry):
        g = 2 * p
        for b in range(2):
            fire_g(g + b + 1, 1 - b)
            drain_g(b)
            fire_w(g + b, b)
            drain_w(b)
        return carry

    lax.fori_loop(0, G_NGRP // 2 - 1, pair, 0)
    g = G_NGRP - 2
    fire_g(g + 1, 1)
    drain_g(0)
    fire_w(g, 0)
    drain_w(0)
    drain_g(1)
    fire_w(g + 1, 1)
    drain_w(1)

    toff = G_NGRP * G_GROUP
    pltpu.async_copy(
        table.at[idx_v.at[pl.ds(toff, G_TAIL)]],
        bufa[0].at[pl.ds(0, G_TAIL)], gsa).wait()
    pltpu.sync_copy(bufa[0].at[pl.ds(0, G_TAIL)],
                    out.at[pl.ds(base + toff, G_TAIL)])


S_NCHUNK = EH // S_CH
S_BASE = S_NCHUNK // NW
S_REM = S_NCHUNK - S_BASE * NW

@functools.partial(
    pl.kernel,
    out_type=jax.ShapeDtypeStruct((NC * N_NODES, D), jnp.float32),
    mesh=_mesh,
    scratch_types=[
        [pltpu.VMEM((S_CH,), jnp.int32)] * 2,
        [pltpu.VMEM((S_CH, D), jnp.float32)] * 2,
        pltpu.VMEM_SHARED((N_NODES, D), jnp.float32),
        pltpu.SemaphoreType.DMA,
        pltpu.SemaphoreType.DMA,
        pltpu.SemaphoreType.DMA,
        pltpu.SemaphoreType.DMA,
    ],
)
def _scatter(rows_hbm, dst1, zeros_hbm, out, idx_c, rv, acc, is0, is1,
             ls0, ls1):
    c = lax.axis_index("c")
    s = lax.axis_index("s")
    wid = s * NC + c
    isem = (is0, is1)
    lsem = (ls0, ls1)

    nch = S_BASE + jnp.where(wid < S_REM, 1, 0)
    cbase = wid * S_BASE + jnp.minimum(wid, S_REM)

    def fire(j, par):
        goff = pl.multiple_of((cbase + j) * S_CH, S_CH)
        pltpu.async_copy(dst1.at[pl.ds(goff, S_CH)], idx_c[par], isem[par])
        pltpu.async_copy(rows_hbm.at[pl.ds(goff, S_CH)], rv[par], lsem[par])

    def wait(par):
        pltpu.make_async_copy(dst1.at[pl.ds(0, S_CH)], idx_c[par],
                              isem[par]).wait()
        pltpu.make_async_copy(rows_hbm.at[pl.ds(0, S_CH)], rv[par],
                              lsem[par]).wait()

    def add(par):
        pltpu.sync_copy(rv[par], acc.at[idx_c[par]], add=True)

    fire(0, 0)
    zb = pl.multiple_of(s * ZR, 8)
    pltpu.sync_copy(zeros_hbm.at[pl.ds(zb, ZR)], acc.at[pl.ds(zb, ZR)])

    @pl.when(s == NS - 1)
    def _():
        tb = pl.multiple_of(NS * ZR, 8)
        pltpu.sync_copy(zeros_hbm.at[pl.ds(tb, ZTAIL)],
                        acc.at[pl.ds(tb, ZTAIL)])

    plsc.subcore_barrier()

    def pair(p, carry):
        j = 2 * p
        for b in range(2):
            wait(b)
            fire(j + b + 1, 1 - b)
            add(b)
        return carry

    lax.fori_loop(0, (S_BASE - 1) // 2, pair, 0)
    wait(0)

    @pl.when(wid < S_REM)
    def _():
        fire(S_BASE, 1)

    add(0)

    @pl.when(wid < S_REM)
    def _():
        wait(1)
        add(1)

    plsc.subcore_barrier()
    zo = pl.multiple_of(c * N_NODES + s * ZR, 8)
    pltpu.sync_copy(acc.at[pl.ds(zb, ZR)], out.at[pl.ds(zo, ZR)])

    @pl.when(s == NS - 1)
    def _():
        tb = pl.multiple_of(NS * ZR, 8)
        to = pl.multiple_of(c * N_NODES + NS * ZR, 8)
        pltpu.sync_copy(acc.at[pl.ds(tb, ZTAIL)], out.at[pl.ds(to, ZTAIL)])


BE = 5000
BN = 2000
_PREC = lax.Precision.DEFAULT


def _full(shape):
    return pl.BlockSpec(shape, lambda i: tuple(0 for _ in shape))


def _edge_body(g1, g2, he, we, we2, b1, b2, gg, bb, out):
    hev = he[...]
    x = g1[...] + g2[...] + b1[...] + jnp.dot(
        hev, we[...], preferred_element_type=jnp.float32, precision=_PREC)
    h = jnp.maximum(x, 0.0)
    e = b2[...] + jnp.dot(
        h, we2[...], preferred_element_type=jnp.float32, precision=_PREC)
    m = jnp.mean(e, axis=-1, keepdims=True)
    v = jnp.mean((e - m) ** 2, axis=-1, keepdims=True)
    e = (e - m) * lax.rsqrt(v + 1e-5) * gg[...] + bb[...]
    out[...] = hev + e


def _edge_call(G, he, we, we2, b1, b2, gg, bb):
    nblk = EH // BE
    return pl.pallas_call(
        _edge_body,
        grid=(nblk,),
        in_specs=[
            pl.BlockSpec((BE, D), lambda i: (i, 0)),
            pl.BlockSpec((BE, D), lambda i: (nblk + i, 0)),
            pl.BlockSpec((BE, D), lambda i: (i, 0)),
            _full((D, D)), _full((D, D)),
            _full((1, D)), _full((1, D)), _full((1, D)), _full((1, D)),
        ],
        out_specs=pl.BlockSpec((BE, D), lambda i: (i, 0)),
        out_shape=jax.ShapeDtypeStruct((EH, D), jnp.float32),
        compiler_params=pltpu.CompilerParams(
            dimension_semantics=("arbitrary",)),
    )(G, G, he, we, we2, b1, b2, gg, bb)


def _node_core(hv, agg, w1a, w1b, b1, w2, b2, gg, bb):
    x = b1[...] + jnp.dot(
        hv, w1a[...], preferred_element_type=jnp.float32, precision=_PREC)
    x = x + jnp.dot(
        agg, w1b[...], preferred_element_type=jnp.float32, precision=_PREC)
    h = jnp.maximum(x, 0.0)
    n = b2[...] + jnp.dot(
        h, w2[...], preferred_element_type=jnp.float32, precision=_PREC)
    m = jnp.mean(n, axis=-1, keepdims=True)
    v = jnp.mean((n - m) ** 2, axis=-1, keepdims=True)
    n = (n - m) * lax.rsqrt(v + 1e-5) * gg[...] + bb[...]
    return hv + n


def _agg4(a0a, a0b, a1a, a1b):
    return (a0a[...] + a0b[...]) + (a1a[...] + a1b[...])


def _node_body_p(hn, a0a, a0b, a1a, a1b, w1a, w1b, b1, w2, b2, gg, bb, wnx,
                 hout, pout):
    hv = hn[...]
    hnew = _node_core(hv, _agg4(a0a, a0b, a1a, a1b),
                      w1a, w1b, b1, w2, b2, gg, bb)
    hout[...] = hnew
    pout[0] = jnp.dot(hnew, wnx[0],
                      preferred_element_type=jnp.float32, precision=_PREC)
    pout[1] = jnp.dot(hnew, wnx[1],
                      preferred_element_type=jnp.float32, precision=_PREC)


def _node_body(hn, a0a, a0b, a1a, a1b, w1a, w1b, b1, w2, b2, gg, bb, hout):
    hv = hn[...]
    hout[...] = _node_core(hv, _agg4(a0a, a0b, a1a, a1b),
                           w1a, w1b, b1, w2, b2, gg, bb)


def _node_specs():
    nblk = N_NODES // BN
    return [
        pl.BlockSpec((BN, D), lambda i: (i, 0)),
        pl.BlockSpec((BN, D), lambda i: (i, 0)),
        pl.BlockSpec((BN, D), lambda i: (nblk + i, 0)),
        pl.BlockSpec((BN, D), lambda i: (i, 0)),
        pl.BlockSpec((BN, D), lambda i: (nblk + i, 0)),
        _full((D, D)), _full((D, D)), _full((1, D)),
        _full((D, D)), _full((1, D)), _full((1, D)), _full((1, D)),
    ]


def _node_call_p(hn, a0, a1, w1a, w1b, b1, w2, b2, gg, bb, wnx):
    return pl.pallas_call(
        _node_body_p,
        grid=(N_NODES // BN,),
        in_specs=_node_specs() + [_full((2, D, D))],
        out_specs=[
            pl.BlockSpec((BN, D), lambda i: (i, 0)),
            pl.BlockSpec((2, BN, D), lambda i: (0, i, 0)),
        ],
        out_shape=[
            jax.ShapeDtypeStruct((N_NODES, D), jnp.float32),
            jax.ShapeDtypeStruct((2, N_NODES, D), jnp.float32),
        ],
        compiler_params=pltpu.CompilerParams(
            dimension_semantics=("arbitrary",)),
    )(hn, a0, a0, a1, a1, w1a, w1b, b1, w2, b2, gg, bb, wnx)


def _node_call(hn, a0, a1, w1a, w1b, b1, w2, b2, gg, bb):
    return pl.pallas_call(
        _node_body,
        grid=(N_NODES // BN,),
        in_specs=_node_specs(),
        out_specs=pl.BlockSpec((BN, D), lambda i: (i, 0)),
        out_shape=jax.ShapeDtypeStruct((N_NODES, D), jnp.float32),
        compiler_params=pltpu.CompilerParams(
            dimension_semantics=("arbitrary",)),
    )(hn, a0, a0, a1, a1, w1a, w1b, b1, w2, b2, gg, bb)


def _p0_body(hn, wnx, pout):
    hv = hn[...]
    pout[0] = jnp.dot(hv, wnx[0],
                      preferred_element_type=jnp.float32, precision=_PREC)
    pout[1] = jnp.dot(hv, wnx[1],
                      preferred_element_type=jnp.float32, precision=_PREC)


def _p0_call(hn, wnx):
    return pl.pallas_call(
        _p0_body,
        grid=(N_NODES // BN,),
        in_specs=[pl.BlockSpec((BN, D), lambda i: (i, 0)), _full((2, D, D))],
        out_specs=pl.BlockSpec((2, BN, D), lambda i: (0, i, 0)),
        out_shape=jax.ShapeDtypeStruct((2, N_NODES, D), jnp.float32),
        compiler_params=pltpu.CompilerParams(
            dimension_semantics=("arbitrary",)),
    )(hn, wnx)


def kernel(h_node, h_edge, edge_index, We1, be1, We2, be2, ge, bbe,
           Wn1, bn1, Wn2, bn2, gn, bbn):
    src = edge_index[0].astype(jnp.int32)
    dst = edge_index[1].astype(jnp.int32)
    dst_h = (dst[:EH], dst[EH:])
    idx_h = (jnp.concatenate([src[:EH], dst_h[0] + N_NODES]),
             jnp.concatenate([src[EH:], dst_h[1] + N_NODES]))
    he = [h_edge[:EH], h_edge[EH:]]
    zeros = jnp.zeros((N_NODES, D), jnp.float32)

    P = _p0_call(h_node, We1[0, :2 * D].reshape(2, D, D))
    for l in range(NUM_CONVS):
        Pflat = P.reshape(2 * N_NODES, D)
        G = [_gather(Pflat, idx_h[h]) for h in range(2)]
        agg = [None, None]
        for h in range(2):
            he[h] = _edge_call(G[h], he[h], We1[l, 2 * D:], We2[l],
                               be1[l][None], be2[l][None],
                               ge[l][None], bbe[l][None])
            agg[h] = _scatter(he[h], dst_h[h], zeros)
        args = (h_node, agg[0], agg[1], Wn1[l, :D], Wn1[l, D:], bn1[l][None],
                Wn2[l], bn2[l][None], gn[l][None], bbn[l][None])
        if l + 1 < NUM_CONVS:
            h_node, P = _node_call_p(*args, We1[l + 1, :2 * D].reshape(2, D, D))
        else:
            h_node = _node_call(*args)
    return (h_node, jnp.concatenate(he))
